# Initial kernel scaffold; baseline (speedup 1.0000x reference)
import functools

import jax
import jax.numpy as jnp
from jax import lax
from jax.experimental import pallas as pl
from jax.experimental.pallas import tpu as pltpu
from jax.experimental.pallas import tpu_sc as plsc

_NN = 10000
_NE = 20000
_NNZ = 320000
_DH = 256

_NSC = 2
_NSUB = 16
_G = 40
_NBUF = 4
_PER_TILE = _NNZ // _NSUB
_NB = _PER_TILE // _G
_NNP = 10240
_NEP = 20480

_mesh = plsc.VectorSubcoreMesh(core_axis_name="c", subcore_axis_name="s")


def _zero_rows(zeros_hbm, acc, r0, nr):
    pltpu.sync_copy(zeros_hbm.at[pl.ds(r0, nr)], acc.at[pl.ds(r0, nr)])


def _grouped_scatter_add(idxd_v, acc, src_of, ssems):
    @pl.loop(0, _NB // _NBUF)
    def _(i):
        b0 = i * _NBUF
        descs = []
        for k in range(_NBUF):
            d = pltpu.async_copy(src_of(b0 + k),
                                 acc.at[idxd_v.at[b0 + k]],
                                 ssems.at[k], add=True)
            descs.append(d)
        for d in descs:
            d.wait()


def _sc_counts(nidx3, eidx3, zeros1):
    @functools.partial(
        pl.kernel,
        out_type=(jax.ShapeDtypeStruct((_NNP,), jnp.float32),
                  jax.ShapeDtypeStruct((_NEP,), jnp.float32)),
        mesh=_mesh,
        scratch_types=[
            pltpu.VMEM((_NB, _G), jnp.int32),
            pltpu.VMEM((48,), jnp.float32),
            pltpu.VMEM_SHARED((_NNP,), jnp.float32),
            pltpu.VMEM_SHARED((_NEP,), jnp.float32),
            pltpu.SemaphoreType.DMA((_NBUF,)),
        ],
    )
    def k(nidx_h, eidx_h, zeros_h, deg_o, card_o, idxd_v, ones_v,
          accn, acce, ssems):
        c = lax.axis_index("c")
        s = lax.axis_index("s")
        for v in range(3):
            ones_v[pl.ds(v * 16, 16)] = jnp.ones((16,), jnp.float32)

        def path(idx3_h, acc, out_h, nr):
            pltpu.sync_copy(idx3_h.at[s], idxd_v)
            _zero_rows(zeros_h, acc, s * nr, nr)
            plsc.subcore_barrier()
            _grouped_scatter_add(
                idxd_v, acc, lambda b: ones_v.at[pl.ds(0, _G)], ssems)
            plsc.subcore_barrier()
            pltpu.sync_copy(acc.at[pl.ds(s * nr, nr)],
                            out_h.at[pl.ds(s * nr, nr)])

        @pl.when(c == 0)
        def _():
            path(nidx_h, accn, deg_o, _NNP // _NSUB)

        @pl.when(c == 1)
        def _():
            path(eidx_h, acce, card_o, _NEP // _NSUB)

    return k(nidx3, eidx3, zeros1)


def _sc_wsums(nidx3, eidx3, nflat, eflat, d0r, d1r, zeros1):
    @functools.partial(
        pl.kernel,
        out_type=(jax.ShapeDtypeStruct((_NNP,), jnp.float32),
                  jax.ShapeDtypeStruct((_NEP,), jnp.float32)),
        mesh=_mesh,
        scratch_types=[
            pltpu.VMEM((_NB, _G), jnp.int32),
            pltpu.VMEM((_PER_TILE,), jnp.int32),
            pltpu.VMEM((_PER_TILE,), jnp.float32),
            pltpu.VMEM((_NEP,), jnp.float32),
            pltpu.VMEM_SHARED((_NNP,), jnp.float32),
            pltpu.VMEM_SHARED((_NEP,), jnp.float32),
            pltpu.SemaphoreType.DMA((_NBUF,)),
        ],
    )
    def k(nidx_h, eidx_h, nflat_h, eflat_h, d0r_h, d1r_h, zeros_h,
          sn_o, se_o, idxd_v, idxs_v, val_v, tab_v, accn, acce, ssems):
        c = lax.axis_index("c")
        s = lax.axis_index("s")

        def path(dst3_h, srcflat_h, tab_h, tabn, acc, out_h, nr):
            pltpu.sync_copy(dst3_h.at[s], idxd_v)
            pltpu.sync_copy(srcflat_h.at[s], idxs_v)
            pltpu.sync_copy(tab_h, tab_v.at[pl.ds(0, tabn)])
            _zero_rows(zeros_h, acc, s * nr, nr)

            @pl.loop(0, _PER_TILE // 16)
            def _(v):
                sl = pl.ds(v * 16, 16)
                val_v[sl] = plsc.load_gather(tab_v, [idxs_v[sl]])

            plsc.subcore_barrier()
            _grouped_scatter_add(
                idxd_v, acc, lambda b: val_v.at[pl.ds(b * _G, _G)], ssems)
            plsc.subcore_barrier()
            pltpu.sync_copy(acc.at[pl.ds(s * nr, nr)],
                            out_h.at[pl.ds(s * nr, nr)])

        @pl.when(c == 0)
        def _():
            path(nidx_h, eflat_h, d1r_h, _NEP, accn, sn_o, _NNP // _NSUB)

        @pl.when(c == 1)
        def _():
            path(eidx_h, nflat_h, d0r_h, _NNP, acce, se_o, _NEP // _NSUB)

    return k(nidx3, eidx3, nflat, eflat, d0r, d1r, zeros1)


def _sc_seg_rows(tab_flat, srcflat, dst3, zeros, nslab, w, r_tab, s_out):
    npass = nslab // _NSC
    nr = s_out // _NSUB

    @functools.partial(
        pl.kernel,
        out_type=jax.ShapeDtypeStruct((nslab, s_out, w), jnp.float32),
        mesh=_mesh,
        scratch_types=[
            pltpu.VMEM((_PER_TILE,), jnp.int32),
            pltpu.VMEM((_PER_TILE,), jnp.int32),
            pltpu.VMEM((_NB, _G), jnp.int32),
            pltpu.VMEM((_NBUF, _G, w), jnp.float32),
            pltpu.VMEM_SHARED((s_out, w), jnp.float32),
            pltpu.SemaphoreType.DMA((_NBUF,)),
            pltpu.SemaphoreType.DMA((_NBUF,)),
        ],
    )
    def k(tab_h, src_h, dst_h, zeros_h, out_h,
          idxs_v, idxg_v, idxd_v, gbuf, acc, gsems, ssems):
        c = lax.axis_index("c")
        s = lax.axis_index("s")
        pltpu.sync_copy(src_h.at[s], idxs_v)
        pltpu.sync_copy(dst_h.at[s], idxd_v)
        for p in range(npass):
            slab = c * npass + p
            off = slab * r_tab

            @pl.loop(0, _PER_TILE // 16)
            def _(v):
                sl = pl.ds(v * 16, 16)
                idxg_v[sl] = idxs_v[sl] + off

            _zero_rows(zeros_h, acc, s * nr, nr)
            plsc.subcore_barrier()

            @pl.loop(0, _NB // _NBUF)
            def _(i):
                b0 = i * _NBUF
                gds = []
                for kk in range(_NBUF):
                    gds.append(pltpu.async_copy(
                        tab_h.at[idxg_v.at[pl.ds((b0 + kk) * _G, _G)]],
                        gbuf.at[kk], gsems.at[kk]))
                sds = []
                for kk in range(_NBUF):
                    gds[kk].wait()
                    sds.append(pltpu.async_copy(
                        gbuf.at[kk], acc.at[idxd_v.at[b0 + kk]],
                        ssems.at[kk], add=True))
                for d in sds:
                    d.wait()

            plsc.subcore_barrier()
            pltpu.sync_copy(acc.at[pl.ds(s * nr, nr)],
                            out_h.at[slab].at[pl.ds(s * nr, nr)])
            plsc.subcore_barrier()

    return k(tab_flat, srcflat, dst3, zeros)


def _tc_powers(deg2, card2):
    def body(d_ref, c_ref, d0_ref, d1_ref):
        d = jnp.maximum(d_ref[...], 1.0)
        d0_ref[...] = lax.rsqrt(d)
        m = jnp.maximum(c_ref[...], 1.0)
        r = lax.rsqrt(m)
        d1_ref[...] = r * r * r

    return pl.pallas_call(
        body,
        out_shape=(jax.ShapeDtypeStruct(deg2.shape, jnp.float32),
                   jax.ShapeDtypeStruct(card2.shape, jnp.float32)),
    )(deg2, card2)


def _tc_stage(x, w_mat, bias, f_raw, g_row, nslab_in, nslab_out, w_out,
              rb):
    if nslab_in > 1:
        _, r, wi = x.shape
    else:
        r, wi = x.shape
    grid = (r // rb,)

    def body(*refs):
        i = 0
        x_ref = refs[i]; i += 1
        w_ref = refs[i]; i += 1
        b_ref = f_ref = g_ref = None
        if f_raw is not None:
            f_ref = refs[i]; i += 1
            b_ref = refs[i]; i += 1
        if g_row is not None:
            g_ref = refs[i]; i += 1
        o_ref = refs[i]
        if nslab_in > 1:
            xv = jnp.concatenate([x_ref[ss] for ss in range(nslab_in)],
                                 axis=-1)
        else:
            xv = x_ref[...]
        if f_raw is not None:
            f = 1.0 / jnp.maximum(f_ref[...], 1e-12)
            xv = jnp.maximum(xv * f + b_ref[...], 0.0)
        y = jnp.dot(xv, w_ref[...], preferred_element_type=jnp.float32)
        if g_row is not None:
            y = y * g_ref[...]
        if nslab_out > 1:
            for ss in range(nslab_out):
                o_ref[ss] = y[:, ss * w_out:(ss + 1) * w_out]
        else:
            o_ref[...] = y

    in_specs = []
    args = []
    if nslab_in > 1:
        in_specs.append(pl.BlockSpec((nslab_in, rb, wi),
                                     lambda i: (0, i, 0)))
    else:
        in_specs.append(pl.BlockSpec((rb, wi), lambda i: (i, 0)))
    args.append(x)
    in_specs.append(pl.BlockSpec(w_mat.shape, lambda i: (0, 0)))
    args.append(w_mat)
    if f_raw is not None:
        in_specs.append(pl.BlockSpec((rb, 1), lambda i: (i, 0)))
        args.append(f_raw)
        in_specs.append(pl.BlockSpec((1, _DH), lambda i: (0, 0)))
        args.append(bias.reshape(1, _DH))
    if g_row is not None:
        in_specs.append(pl.BlockSpec((rb, 1), lambda i: (i, 0)))
        args.append(g_row)
    if nslab_out > 1:
        out_shape = jax.ShapeDtypeStruct((nslab_out, r, w_out),
                                         jnp.float32)
        out_spec = pl.BlockSpec((nslab_out, rb, w_out),
                                lambda i: (0, i, 0))
    else:
        out_shape = jax.ShapeDtypeStruct((r, _DH), jnp.float32)
        out_spec = pl.BlockSpec((rb, _DH), lambda i: (i, 0))

    return pl.pallas_call(
        body, grid=grid, in_specs=in_specs, out_specs=out_spec,
        out_shape=out_shape)(*args)


def _tc_head(agg, f_raw, bias, lin_w_row, lin_b, rb):
    ns, r, wi = agg.shape
    grid = (r // rb,)

    def body(x_ref, f_ref, b_ref, lw_ref, lb_ref, o_ref, mx_ref):
        i = pl.program_id(0)
        xv = jnp.concatenate([x_ref[ss] for ss in range(ns)], axis=-1)
        f = 1.0 / jnp.maximum(f_ref[...], 1e-12)
        xv = jnp.maximum(xv * f + b_ref[...], 0.0)
        m = jnp.max(xv, axis=0, keepdims=True)

        @pl.when(i == 0)
        def _():
            mx_ref[...] = m

        @pl.when(i > 0)
        def _():
            mx_ref[...] = jnp.maximum(mx_ref[...], m)

        @pl.when(i == grid[0] - 1)
        def _():
            o_ref[...] = (jnp.sum(mx_ref[...] * lw_ref[...],
                                  axis=-1, keepdims=True) + lb_ref[...])

    return pl.pallas_call(
        body, grid=grid,
        in_specs=[
            pl.BlockSpec((ns, rb, wi), lambda i: (0, i, 0)),
            pl.BlockSpec((rb, 1), lambda i: (i, 0)),
            pl.BlockSpec((1, _DH), lambda i: (0, 0)),
            pl.BlockSpec((1, _DH), lambda i: (0, 0)),
            pl.BlockSpec((1, 1), lambda i: (0, 0)),
        ],
        out_specs=pl.BlockSpec((1, 1), lambda i: (0, 0)),
        out_shape=jax.ShapeDtypeStruct((1, 1), jnp.float32),
        scratch_shapes=[pltpu.VMEM((1, _DH), jnp.float32)],
    )(agg, f_raw, bias.reshape(1, _DH), lin_w_row, lin_b.reshape(1, 1))


def kernel(x_0, node_idx, edge_idx, W01_0, b1_0, W10_0, b0_0,
           W01_1, b1_1, W10_1, b0_1, lin_W, lin_b):
    node_idx = node_idx.astype(jnp.int32)
    edge_idx = edge_idx.astype(jnp.int32)
    nidx3 = node_idx.reshape(_NSUB, _NB, _G)
    eidx3 = edge_idx.reshape(_NSUB, _NB, _G)
    nflat = node_idx.reshape(_NSUB, _PER_TILE)
    eflat = edge_idx.reshape(_NSUB, _PER_TILE)
    zeros_e64 = jnp.zeros((_NE, 64), jnp.float32)
    zeros_n128 = jnp.zeros((_NN, 128), jnp.float32)
    zeros1 = jnp.zeros((_NEP,), jnp.float32)

    deg_p, card_p = _sc_counts(nidx3, eidx3, zeros1)
    d0r_p, d1r_p = _tc_powers(deg_p.reshape(80, 128),
                              card_p.reshape(160, 128))
    d0r_p = d0r_p.reshape(_NNP)
    d1r_p = d1r_p.reshape(_NEP)
    s_node_p, s_edge_p = _sc_wsums(nidx3, eidx3, nflat, eflat,
                                   d0r_p, d1r_p, zeros1)
    d0r = d0r_p[:_NN].reshape(_NN, 1)
    d1r = d1r_p[:_NE].reshape(_NE, 1)
    s_node = s_node_p[:_NN].reshape(_NN, 1)
    s_edge = s_edge_p[:_NE].reshape(_NE, 1)

    t1 = _tc_stage(x_0, W01_0, None, None, d0r, 1, 4, 64, 1000)
    agg1 = _sc_seg_rows(t1.reshape(4 * _NN, 64), nflat, eidx3,
                        zeros_e64, 4, 64, _NN, _NE)
    t2 = _tc_stage(agg1, W10_0, b1_0, s_edge, d1r, 4, 2, 128, 1000)
    agg2 = _sc_seg_rows(t2.reshape(2 * _NE, 128), eflat, nidx3,
                        zeros_n128, 2, 128, _NE, _NN)
    t3 = _tc_stage(agg2, W01_1, b0_0, s_node, d0r, 2, 4, 64, 1000)
    agg3 = _sc_seg_rows(t3.reshape(4 * _NN, 64), nflat, eidx3,
                        zeros_e64, 4, 64, _NN, _NE)
    t4 = _tc_stage(agg3, W10_1, b1_1, s_edge, d1r, 4, 2, 128, 1000)
    agg4 = _sc_seg_rows(t4.reshape(2 * _NE, 128), eflat, nidx3,
                        zeros_n128, 2, 128, _NE, _NN)
    out = _tc_head(agg4, s_node, b0_1, lin_W.reshape(1, _DH), lin_b, 1000)
    return out.reshape(1)


# speedup vs baseline: 10.3275x; 10.3275x over previous
import functools

import jax
import jax.numpy as jnp
from jax import lax
from jax.experimental import pallas as pl
from jax.experimental.pallas import tpu as pltpu
from jax.experimental.pallas import tpu_sc as plsc

_NN = 10000
_NE = 20000
_NNZ = 320000
_DH = 256

_NSC = 2
_NSUB = 16
_G = 40
_NBUF = 4
_PER_TILE = _NNZ // _NSUB
_NB = _PER_TILE // _G
_NNP = 10240
_NEP = 20480

_SC_PARAMS = pltpu.CompilerParams(needs_layout_passes=False,
                                  use_tc_tiling_on_sc=False)


@functools.cache
def _sc_mesh():
    return plsc.VectorSubcoreMesh(core_axis_name="c", subcore_axis_name="s",
                                  num_cores=_NSC, num_subcores=_NSUB)


def _zero_rows(zeros_hbm, acc, r0, nr):
    pltpu.sync_copy(zeros_hbm.at[pl.ds(r0, nr)], acc.at[pl.ds(r0, nr)])


def _grouped_scatter_add(idxd_v, acc, src_of, ssems):
    @pl.loop(0, _NB // _NBUF)
    def _(i):
        b0 = i * _NBUF
        descs = []
        for k in range(_NBUF):
            d = pltpu.async_copy(src_of(b0 + k),
                                 acc.at[idxd_v.at[b0 + k]],
                                 ssems.at[k], add=True)
            descs.append(d)
        for d in descs:
            d.wait()


def _sc_counts(nidx3, eidx3, zeros1):
    @functools.partial(
        pl.kernel,
        out_type=(jax.ShapeDtypeStruct((_NNP,), jnp.float32),
                  jax.ShapeDtypeStruct((_NEP,), jnp.float32)),
        mesh=_sc_mesh(),
        compiler_params=_SC_PARAMS,
        scratch_types=[
            pltpu.VMEM((_NB, _G), jnp.int32),
            pltpu.VMEM((48,), jnp.float32),
            pltpu.VMEM_SHARED((_NNP,), jnp.float32),
            pltpu.VMEM_SHARED((_NEP,), jnp.float32),
            pltpu.SemaphoreType.DMA((_NBUF,)),
        ],
    )
    def k(nidx_h, eidx_h, zeros_h, deg_o, card_o, idxd_v, ones_v,
          accn, acce, ssems):
        c = lax.axis_index("c")
        s = lax.axis_index("s")
        for v in range(3):
            ones_v[pl.ds(v * 16, 16)] = jnp.ones((16,), jnp.float32)

        def path(idx3_h, acc, out_h, nr):
            pltpu.sync_copy(idx3_h.at[s], idxd_v)
            _zero_rows(zeros_h, acc, s * nr, nr)
            plsc.subcore_barrier()
            _grouped_scatter_add(
                idxd_v, acc, lambda b: ones_v.at[pl.ds(0, _G)], ssems)
            plsc.subcore_barrier()
            pltpu.sync_copy(acc.at[pl.ds(s * nr, nr)],
                            out_h.at[pl.ds(s * nr, nr)])

        @pl.when(c == 0)
        def _():
            path(nidx_h, accn, deg_o, _NNP // _NSUB)

        @pl.when(c == 1)
        def _():
            path(eidx_h, acce, card_o, _NEP // _NSUB)

    return k(nidx3, eidx3, zeros1)


def _sc_wsums(nidx3, eidx3, nflat, eflat, d0r, d1r, zeros1):
    @functools.partial(
        pl.kernel,
        out_type=(jax.ShapeDtypeStruct((_NNP,), jnp.float32),
                  jax.ShapeDtypeStruct((_NEP,), jnp.float32)),
        mesh=_sc_mesh(),
        compiler_params=_SC_PARAMS,
        scratch_types=[
            pltpu.VMEM((_NB, _G), jnp.int32),
            pltpu.VMEM((_PER_TILE,), jnp.int32),
            pltpu.VMEM((_PER_TILE,), jnp.float32),
            pltpu.VMEM((_NEP,), jnp.float32),
            pltpu.VMEM_SHARED((_NNP,), jnp.float32),
            pltpu.VMEM_SHARED((_NEP,), jnp.float32),
            pltpu.SemaphoreType.DMA((_NBUF,)),
        ],
    )
    def k(nidx_h, eidx_h, nflat_h, eflat_h, d0r_h, d1r_h, zeros_h,
          sn_o, se_o, idxd_v, idxs_v, val_v, tab_v, accn, acce, ssems):
        c = lax.axis_index("c")
        s = lax.axis_index("s")

        def path(dst3_h, srcflat_h, tab_h, tabn, acc, out_h, nr):
            pltpu.sync_copy(dst3_h.at[s], idxd_v)
            pltpu.sync_copy(srcflat_h.at[s], idxs_v)
            pltpu.sync_copy(tab_h, tab_v.at[pl.ds(0, tabn)])
            _zero_rows(zeros_h, acc, s * nr, nr)

            @pl.loop(0, _PER_TILE // 16)
            def _(v):
                sl = pl.ds(v * 16, 16)
                val_v[sl] = plsc.load_gather(tab_v, [idxs_v[sl] - 1])

            plsc.subcore_barrier()
            _grouped_scatter_add(
                idxd_v, acc, lambda b: val_v.at[pl.ds(b * _G, _G)], ssems)
            plsc.subcore_barrier()
            pltpu.sync_copy(acc.at[pl.ds(s * nr, nr)],
                            out_h.at[pl.ds(s * nr, nr)])

        @pl.when(c == 0)
        def _():
            path(nidx_h, eflat_h, d1r_h, _NEP, accn, sn_o, _NNP // _NSUB)

        @pl.when(c == 1)
        def _():
            path(eidx_h, nflat_h, d0r_h, _NNP, acce, se_o, _NEP // _NSUB)

    return k(nidx3, eidx3, nflat, eflat, d0r, d1r, zeros1)


def _sc_seg_rows(tab_flat, srcflat, dst3, zeros, nslab, w, r_tab, s_pad):
    npass = nslab // _NSC
    nr = s_pad // _NSUB
    nbuf = 5

    @functools.partial(
        pl.kernel,
        out_type=jax.ShapeDtypeStruct((nslab, s_pad, w), jnp.float32),
        mesh=_sc_mesh(),
        compiler_params=_SC_PARAMS,
        scratch_types=[
            pltpu.VMEM((_PER_TILE // 2,), jnp.int32),
            pltpu.VMEM((_NB // 2, _G), jnp.int32),
            pltpu.VMEM((nbuf, _G, w), jnp.float32),
            pltpu.VMEM_SHARED((s_pad, w), jnp.float32),
            pltpu.SemaphoreType.DMA((nbuf,)),
            pltpu.SemaphoreType.DMA((nbuf,)),
        ],
    )
    def k(tab_h, src_h, dst_h, zeros_h, out_h,
          idxg_v, idxd_v, gbuf, acc, gsems, ssems):
        c = lax.axis_index("c")
        s = lax.axis_index("s")
        hlen = _PER_TILE // 2
        hnb = _NB // 2
        for p in range(npass):
            slab = c * npass + p
            off = slab * r_tab - 1

            _zero_rows(zeros_h, acc, s * nr, nr)
            plsc.subcore_barrier()

            for h in range(2):
                pltpu.sync_copy(src_h.at[s].at[pl.ds(h * hlen, hlen)],
                                idxg_v)
                pltpu.sync_copy(dst_h.at[s].at[pl.ds(h * hnb, hnb)],
                                idxd_v)

                @pl.loop(0, hlen // 16)
                def _(v):
                    sl = pl.ds(v * 16, 16)
                    idxg_v[sl] = idxg_v[sl] + off

                @pl.loop(0, hnb // nbuf)
                def _(i):
                    b0 = i * nbuf
                    gds = []
                    for kk in range(nbuf):
                        gds.append(pltpu.async_copy(
                            tab_h.at[idxg_v.at[pl.ds((b0 + kk) * _G, _G)]],
                            gbuf.at[kk], gsems.at[kk]))
                    sds = []
                    for kk in range(nbuf):
                        gds[kk].wait()
                        sds.append(pltpu.async_copy(
                            gbuf.at[kk], acc.at[idxd_v.at[b0 + kk]],
                            ssems.at[kk], add=True))
                    for d in sds:
                        d.wait()

            plsc.subcore_barrier()
            pltpu.sync_copy(acc.at[pl.ds(s * nr, nr)],
                            out_h.at[slab].at[pl.ds(s * nr, nr)])
            plsc.subcore_barrier()

    return k(tab_flat, srcflat, dst3, zeros)


def _tc_powers(deg2, card2):
    def body(d_ref, c_ref, d0_ref, d1_ref):
        d = jnp.maximum(d_ref[...], 1.0)
        d0_ref[...] = lax.rsqrt(d)
        m = jnp.maximum(c_ref[...], 1.0)
        r = lax.rsqrt(m)
        d1_ref[...] = r * r * r

    return pl.pallas_call(
        body,
        out_shape=(jax.ShapeDtypeStruct(deg2.shape, jnp.float32),
                   jax.ShapeDtypeStruct(card2.shape, jnp.float32)),
    )(deg2, card2)


def _tc_stage(x, w_mat, bias, f_raw, g_row, nslab_in, nslab_out, w_out,
              rb, r):
    wi = x.shape[-1]
    grid = (r // rb,)

    def body(*refs):
        i = 0
        x_ref = refs[i]; i += 1
        w_ref = refs[i]; i += 1
        b_ref = f_ref = g_ref = None
        if f_raw is not None:
            f_ref = refs[i]; i += 1
            b_ref = refs[i]; i += 1
        if g_row is not None:
            g_ref = refs[i]; i += 1
        o_ref = refs[i]
        if nslab_in > 1:
            xv = jnp.concatenate([x_ref[ss] for ss in range(nslab_in)],
                                 axis=-1)
        else:
            xv = x_ref[...]
        if f_raw is not None:
            f = 1.0 / jnp.maximum(f_ref[...], 1e-12)
            xv = jnp.maximum(xv * f + b_ref[...], 0.0)
        y = jnp.dot(xv, w_ref[...], preferred_element_type=jnp.float32)
        if g_row is not None:
            y = y * g_ref[...]
        if nslab_out > 1:
            for ss in range(nslab_out):
                o_ref[ss] = y[:, ss * w_out:(ss + 1) * w_out]
        else:
            o_ref[...] = y

    in_specs = []
    args = []
    if nslab_in > 1:
        in_specs.append(pl.BlockSpec((nslab_in, rb, wi),
                                     lambda i: (0, i, 0)))
    else:
        in_specs.append(pl.BlockSpec((rb, wi), lambda i: (i, 0)))
    args.append(x)
    in_specs.append(pl.BlockSpec(w_mat.shape, lambda i: (0, 0)))
    args.append(w_mat)
    if f_raw is not None:
        in_specs.append(pl.BlockSpec((rb, 1), lambda i: (i, 0)))
        args.append(f_raw)
        in_specs.append(pl.BlockSpec((1, _DH), lambda i: (0, 0)))
        args.append(bias.reshape(1, _DH))
    if g_row is not None:
        in_specs.append(pl.BlockSpec((rb, 1), lambda i: (i, 0)))
        args.append(g_row)
    if nslab_out > 1:
        out_shape = jax.ShapeDtypeStruct((nslab_out, r, w_out),
                                         jnp.float32)
        out_spec = pl.BlockSpec((nslab_out, rb, w_out),
                                lambda i: (0, i, 0))
    else:
        out_shape = jax.ShapeDtypeStruct((r, _DH), jnp.float32)
        out_spec = pl.BlockSpec((rb, _DH), lambda i: (i, 0))

    return pl.pallas_call(
        body, grid=grid, in_specs=in_specs, out_specs=out_spec,
        out_shape=out_shape)(*args)


def _tc_head(agg, f_raw, bias, lin_w_row, lin_b, rb, r):
    ns, wi = agg.shape[0], agg.shape[-1]
    grid = (r // rb,)

    def body(x_ref, f_ref, b_ref, lw_ref, lb_ref, o_ref, mx_ref):
        i = pl.program_id(0)
        xv = jnp.concatenate([x_ref[ss] for ss in range(ns)], axis=-1)
        f = 1.0 / jnp.maximum(f_ref[...], 1e-12)
        xv = jnp.maximum(xv * f + b_ref[...], 0.0)
        m = jnp.max(xv, axis=0, keepdims=True)

        @pl.when(i == 0)
        def _():
            mx_ref[...] = m

        @pl.when(i > 0)
        def _():
            mx_ref[...] = jnp.maximum(mx_ref[...], m)

        @pl.when(i == grid[0] - 1)
        def _():
            o_ref[...] = (jnp.sum(mx_ref[...] * lw_ref[...],
                                  axis=-1, keepdims=True) + lb_ref[...])

    return pl.pallas_call(
        body, grid=grid,
        in_specs=[
            pl.BlockSpec((ns, rb, wi), lambda i: (0, i, 0)),
            pl.BlockSpec((rb, 1), lambda i: (i, 0)),
            pl.BlockSpec((1, _DH), lambda i: (0, 0)),
            pl.BlockSpec((1, _DH), lambda i: (0, 0)),
            pl.BlockSpec((1, 1), lambda i: (0, 0)),
        ],
        out_specs=pl.BlockSpec((1, 1), lambda i: (0, 0)),
        out_shape=jax.ShapeDtypeStruct((1, 1), jnp.float32),
        scratch_shapes=[pltpu.VMEM((1, _DH), jnp.float32)],
    )(agg, f_raw, bias.reshape(1, _DH), lin_w_row, lin_b.reshape(1, 1))


def kernel(x_0, node_idx, edge_idx, W01_0, b1_0, W10_0, b0_0,
           W01_1, b1_1, W10_1, b0_1, lin_W, lin_b):
    node_idx = node_idx.astype(jnp.int32)
    edge_idx = edge_idx.astype(jnp.int32)
    nidx3 = node_idx.reshape(_NSUB, _NB, _G)
    eidx3 = edge_idx.reshape(_NSUB, _NB, _G)
    nflat = (node_idx + 1).reshape(_NSUB, _PER_TILE)
    eflat = (edge_idx + 1).reshape(_NSUB, _PER_TILE)
    zeros_e64 = jnp.zeros((_NEP, 64), jnp.float32)
    zeros_n128 = jnp.zeros((_NNP, 128), jnp.float32)
    zeros1 = jnp.zeros((_NEP,), jnp.float32)

    deg_p, card_p = _sc_counts(nidx3, eidx3, zeros1)
    d0r_p, d1r_p = _tc_powers(deg_p.reshape(80, 128),
                              card_p.reshape(160, 128))
    d0r_p = d0r_p.reshape(_NNP)
    d1r_p = d1r_p.reshape(_NEP)
    s_node_p, s_edge_p = _sc_wsums(nidx3, eidx3, nflat, eflat,
                                   d0r_p, d1r_p, zeros1)
    d0r = d0r_p[:_NN].reshape(_NN, 1)
    d1r = d1r_p[:_NE].reshape(_NE, 1)
    s_node = s_node_p[:_NN].reshape(_NN, 1)
    s_edge = s_edge_p[:_NE].reshape(_NE, 1)

    t1 = _tc_stage(x_0, W01_0, None, None, d0r, 1, 4, 64, 1000, _NN)
    agg1 = _sc_seg_rows(t1.reshape(4 * _NN, 64), nflat, eidx3,
                        zeros_e64, 4, 64, _NN, _NEP)
    t2 = _tc_stage(agg1, W10_0, b1_0, s_edge, d1r, 4, 2, 128, 1000, _NE)
    agg2 = _sc_seg_rows(t2.reshape(2 * _NE, 128), eflat, nidx3,
                        zeros_n128, 2, 128, _NE, _NNP)
    t3 = _tc_stage(agg2, W01_1, b0_0, s_node, d0r, 2, 4, 64, 1000, _NN)
    agg3 = _sc_seg_rows(t3.reshape(4 * _NN, 64), nflat, eidx3,
                        zeros_e64, 4, 64, _NN, _NEP)
    t4 = _tc_stage(agg3, W10_1, b1_1, s_edge, d1r, 4, 2, 128, 1000, _NE)
    agg4 = _sc_seg_rows(t4.reshape(2 * _NE, 128), eflat, nidx3,
                        zeros_n128, 2, 128, _NE, _NNP)
    out = _tc_head(agg4, s_node, b0_1, lin_W.reshape(1, _DH), lin_b, 1000, _NN)
    return out.reshape(1)
